# Initial kernel scaffold; baseline (speedup 1.0000x reference)
import jax
import jax.numpy as jnp
from jax.experimental import pallas as pl


def kernel(x, weight):
    raise NotImplementedError("write your pallas kernel here")


# speedup vs baseline: 1.4367x; 1.4367x over previous
import functools

import jax
import jax.numpy as jnp
from jax import lax
from jax.experimental import pallas as pl
from jax.experimental.pallas import tpu as pltpu
from jax.experimental.pallas import tpu_sc as plsc

_D = 32
_CH = 128


@functools.cache
def _make_lookup(B: int, V: int):
    info = plsc.get_sparse_core_info()
    nw = info.num_cores * info.num_subcores
    b_per_w = B // nw
    chunks = b_per_w // _CH
    mesh = plsc.VectorSubcoreMesh(core_axis_name="c", subcore_axis_name="s")

    @functools.partial(
        pl.kernel,
        mesh=mesh,
        out_type=jax.ShapeDtypeStruct((B, _D), jnp.float32),
        scratch_types=[
            pltpu.VMEM((chunks, _CH), jnp.int32),
            pltpu.VMEM((_CH, _D), jnp.float32),
            pltpu.SemaphoreType.DMA,
        ],
        compiler_params=pltpu.CompilerParams(use_tc_tiling_on_sc=False),
    )
    def lookup(idx_hbm, table_hbm, out_hbm, idx_v, rows_v, sem):
        wid = lax.axis_index("s") * info.num_cores + lax.axis_index("c")
        base = wid * b_per_w
        pltpu.sync_copy(idx_hbm.at[wid], idx_v)

        def body(j, carry):
            pltpu.async_copy(table_hbm.at[idx_v.at[j]], rows_v, sem).wait()
            pltpu.sync_copy(rows_v, out_hbm.at[pl.ds(base + j * _CH, _CH)])
            return carry

        lax.fori_loop(0, chunks, body, 0)

    return lookup


def kernel(x, weight):
    B = x.shape[0] * x.shape[1]
    info = plsc.get_sparse_core_info()
    nw = info.num_cores * info.num_subcores
    idx = x.reshape(nw, (B // nw) // _CH, _CH)
    out = _make_lookup(B, weight.shape[0])(idx, weight)
    return out.reshape(x.shape[0], x.shape[1], _D)


# speedup vs baseline: 1.5740x; 1.0955x over previous
import functools

import jax
import jax.numpy as jnp
from jax import lax
from jax.experimental import pallas as pl
from jax.experimental.pallas import tpu as pltpu
from jax.experimental.pallas import tpu_sc as plsc

_D = 32
_CH = 128
_G = 4


@functools.cache
def _make_lookup(B: int, V: int):
    info = plsc.get_sparse_core_info()
    nw = info.num_cores * info.num_subcores
    b_per_w = B // nw
    chunks = b_per_w // _CH
    ng = chunks // _G
    rows_g = _G * _CH
    mesh = plsc.VectorSubcoreMesh(core_axis_name="c", subcore_axis_name="s")

    @functools.partial(
        pl.kernel,
        mesh=mesh,
        out_type=jax.ShapeDtypeStruct((B, _D), jnp.float32),
        scratch_types=[
            pltpu.VMEM((chunks, _CH), jnp.int32),
            pltpu.VMEM((rows_g, _D), jnp.float32),
            pltpu.VMEM((rows_g, _D), jnp.float32),
            pltpu.SemaphoreType.DMA,
            pltpu.SemaphoreType.DMA,
        ],
        compiler_params=pltpu.CompilerParams(use_tc_tiling_on_sc=False),
    )
    def lookup(idx_hbm, table_hbm, out_hbm, idx_v, buf0, buf1, sem0, sem1):
        wid = lax.axis_index("s") * info.num_cores + lax.axis_index("c")
        base = wid * b_per_w
        pltpu.sync_copy(idx_hbm.at[wid], idx_v)

        def fire(g, buf, sem):
            for k in range(_G):
                pltpu.async_copy(
                    table_hbm.at[idx_v.at[g * _G + k]],
                    buf.at[pl.ds(k * _CH, _CH)], sem)

        def drain_write(g, buf, sem):
            pltpu.make_async_copy(table_hbm.at[pl.ds(0, rows_g)], buf, sem).wait()
            pltpu.sync_copy(buf, out_hbm.at[pl.ds(base + g * rows_g, rows_g)])

        fire(0, buf0, sem0)

        def body(t, carry):
            g = 2 * t
            fire(g + 1, buf1, sem1)
            drain_write(g, buf0, sem0)
            fire(g + 2, buf0, sem0)
            drain_write(g + 1, buf1, sem1)
            return carry

        lax.fori_loop(0, ng // 2 - 1, body, 0)
        g_last = ng - 2
        fire(g_last + 1, buf1, sem1)
        drain_write(g_last, buf0, sem0)
        drain_write(g_last + 1, buf1, sem1)

    return lookup


def kernel(x, weight):
    B = x.shape[0] * x.shape[1]
    info = plsc.get_sparse_core_info()
    nw = info.num_cores * info.num_subcores
    idx = x.reshape(nw, (B // nw) // _CH, _CH)
    out = _make_lookup(B, weight.shape[0])(idx, weight)
    return out.reshape(x.shape[0], x.shape[1], _D)
